# baseline (device time: 50526 ns/iter reference)
import os

import jax
import jax.numpy as jnp
from jax import lax
from jax.experimental import pallas as pl
from jax.experimental.pallas import tpu as pltpu

N_DEV = 4
H = 2
_ABLATE = os.environ.get("ABLATE", "")


def kernel(x, w_mat, scale_x, scale_w):
    m_total, k_shard = x.shape
    k_total, n = w_mat.shape
    m_per = m_total // N_DEV
    m_h = m_per // H

    def body(x_ref, w_ref, sx_ref, sw_ref, out_ref,
             x_vmem, send_buf, x_full, w_vmem, w8_ref,
             x_sems, send_sems, recv_sems, w_sems):
        my = lax.axis_index("i")

        bseq = [lax.rem(my + 2, N_DEV),
                lax.rem(my + 1, N_DEV),
                lax.rem(my - 1 + N_DEV, N_DEV),
                my]

        def x_copy(s):
            k, h = divmod(s, H)
            return pltpu.make_async_copy(
                x_ref.at[pl.ds(bseq[k] * m_per + h * m_h, m_h), :],
                x_vmem.at[s % 4],
                x_sems.at[s % 4],
            )

        def w_copy(d):
            return pltpu.make_async_copy(
                w_ref.at[pl.ds(d * m_per, m_per), :],
                w_vmem.at[d % 2],
                w_sems.at[d % 2],
            )

        for s in range(4):
            x_copy(s).start()
        w_copy(0).start()
        w_copy(1).start()

        with jax.named_scope("barrier"):
            barrier = pltpu.get_barrier_semaphore()
            for d in range(1, N_DEV):
                peer = lax.rem(my + d, N_DEV)
                pl.semaphore_signal(barrier, inc=1, device_id=(peer,),
                                    device_id_type=pl.DeviceIdType.MESH)
            pl.semaphore_wait(barrier, N_DEV - 1)

        rdmas = []
        with jax.named_scope("stage_send"):
            for s in range(3 * H):
                k, h = divmod(s, H)
                x_copy(s).wait()
                send_buf[k, pl.ds(h * m_h, m_h), :] = x_vmem[s % 4].astype(
                    jnp.float8_e4m3fn)
                if s + 4 < 4 * H:
                    x_copy(s + 4).start()
                rdma = pltpu.make_async_remote_copy(
                    src_ref=send_buf.at[k, pl.ds(h * m_h, m_h), :],
                    dst_ref=x_full.at[pl.ds(h * m_h, m_h),
                                      pl.ds(my * k_shard, k_shard)],
                    send_sem=send_sems.at[k, h],
                    recv_sem=recv_sems.at[my, h],
                    device_id=(bseq[k],),
                    device_id_type=pl.DeviceIdType.MESH,
                )
                if _ABLATE != "nocomm":
                    rdma.start()
                    rdmas.append(rdma)
            for s in (3 * H, 3 * H + 1):
                h = s % H
                x_copy(s).wait()
                x_full[pl.ds(h * m_h, m_h), pl.ds(my * k_shard, k_shard)] = (
                    x_vmem[s % 4].astype(jnp.float8_e4m3fn))

        for d in range(N_DEV):
            with jax.named_scope(f"wconv#blk={d}"):
                w_copy(d).wait()
                w8_ref[pl.ds(d * m_per, m_per), :] = w_vmem[d % 2].astype(
                    jnp.float8_e5m2)
                if d + 2 < N_DEV:
                    w_copy(d + 2).start()

        for d in range(1, N_DEV):
            src = lax.rem(my + d, N_DEV)
            for h in range(H):
                recv = pltpu.make_async_remote_copy(
                    src_ref=send_buf.at[0, pl.ds(0, m_h), :],
                    dst_ref=x_full.at[pl.ds(h * m_h, m_h),
                                      pl.ds(src * k_shard, k_shard)],
                    send_sem=send_sems.at[0, 0],
                    recv_sem=recv_sems.at[src, h],
                    device_id=(my,),
                    device_id_type=pl.DeviceIdType.MESH,
                )
                with jax.named_scope(f"wait_recv#src={d}_{h}"):
                    if _ABLATE != "nocomm":
                        recv.wait_recv()

        with jax.named_scope("gemm"):
            out_ref[...] = lax.dot_general(
                x_full[...], w8_ref[...], (((1,), (0,)), ((), ())),
                preferred_element_type=jnp.float32,
            ) * (sx_ref[0] * sw_ref[0])

        with jax.named_scope("tail"):
            for rdma in rdmas:
                rdma.wait_send()

    return pl.pallas_call(
        body,
        out_shape=jax.ShapeDtypeStruct((m_per, n), jnp.float32),
        in_specs=[
            pl.BlockSpec(memory_space=pl.ANY),
            pl.BlockSpec(memory_space=pl.ANY),
            pl.BlockSpec(memory_space=pltpu.SMEM),
            pl.BlockSpec(memory_space=pltpu.SMEM),
        ],
        out_specs=pl.BlockSpec(memory_space=pltpu.VMEM),
        scratch_shapes=[
            pltpu.VMEM((4, m_h, k_shard), jnp.float32),
            pltpu.VMEM((N_DEV - 1, m_per, k_shard), jnp.float8_e4m3fn),
            pltpu.VMEM((m_per, k_total), jnp.float8_e4m3fn),
            pltpu.VMEM((2, m_per, n), jnp.float32),
            pltpu.VMEM((k_total, n), jnp.float8_e5m2),
            pltpu.SemaphoreType.DMA((4,)),
            pltpu.SemaphoreType.DMA((N_DEV - 1, H)),
            pltpu.SemaphoreType.DMA((N_DEV, H)),
            pltpu.SemaphoreType.DMA((2,)),
        ],
        compiler_params=pltpu.CompilerParams(
            collective_id=0, vmem_limit_bytes=100 * 1024 * 1024),
    )(x, w_mat, scale_x, scale_w)


# device time: 40441 ns/iter; 1.2494x vs baseline; 1.2494x over previous
import os

import jax
import jax.numpy as jnp
from jax import lax
from jax.experimental import pallas as pl
from jax.experimental.pallas import tpu as pltpu

N_DEV = 4
H = 2
_ABLATE = os.environ.get("ABLATE", "")


def kernel(x, w_mat, scale_x, scale_w):
    m_total, k_shard = x.shape
    k_total, n = w_mat.shape
    m_per = m_total // N_DEV
    m_h = m_per // H

    def body(x_ref, w_ref, sx_ref, sw_ref, out_ref,
             x_vmem, send_buf, recv_buf, w_vmem, w8_ref,
             x_sems, send_sems, recv_sems, w_sems):
        my = lax.axis_index("i")

        bseq = [lax.rem(my - 1 + N_DEV, N_DEV),
                lax.rem(my + 1, N_DEV),
                lax.rem(my + 2, N_DEV),
                my]
        jseq = [my,
                lax.rem(my + 1, N_DEV),
                lax.rem(my - 1 + N_DEV, N_DEV),
                lax.rem(my + 2, N_DEV)]

        def x_copy(s):
            k, h = divmod(s, H)
            return pltpu.make_async_copy(
                x_ref.at[pl.ds(bseq[k] * m_per + h * m_h, m_h), :],
                x_vmem.at[s % 4],
                x_sems.at[s % 4],
            )

        def w_copy(d):
            return pltpu.make_async_copy(
                w_ref.at[pl.ds(jseq[d] * m_per, m_per), :],
                w_vmem.at[d % 2],
                w_sems.at[d % 2],
            )

        for s in range(4):
            x_copy(s).start()

        with jax.named_scope("barrier"):
            barrier = pltpu.get_barrier_semaphore()
            for d in range(1, N_DEV):
                peer = lax.rem(my + d, N_DEV)
                pl.semaphore_signal(barrier, inc=1, device_id=(peer,),
                                    device_id_type=pl.DeviceIdType.MESH)
            pl.semaphore_wait(barrier, N_DEV - 1)

        rdmas = []
        with jax.named_scope("stage_send"):
            for s in range(3 * H):
                k, h = divmod(s, H)
                x_copy(s).wait()
                send_buf[k, pl.ds(h * m_h, m_h), :] = x_vmem[s % 4].astype(
                    jnp.float8_e4m3fn)
                if s + 4 < 4 * H:
                    x_copy(s + 4).start()
                rdma = pltpu.make_async_remote_copy(
                    src_ref=send_buf.at[k, pl.ds(h * m_h, m_h), :],
                    dst_ref=recv_buf.at[my, pl.ds(h * m_h, m_h), :],
                    send_sem=send_sems.at[k, h],
                    recv_sem=recv_sems.at[my, h],
                    device_id=(bseq[k],),
                    device_id_type=pl.DeviceIdType.MESH,
                )
                if _ABLATE != "nocomm":
                    rdma.start()
                    rdmas.append(rdma)
            for s in (3 * H, 3 * H + 1):
                h = s % H
                x_copy(s).wait()
                recv_buf[my, pl.ds(h * m_h, m_h), :] = x_vmem[s % 4].astype(
                    jnp.float8_e4m3fn)

        dot = lambda a, b: lax.dot_general(
            a, b, (((1,), (0,)), ((), ())),
            preferred_element_type=jnp.float32)

        def w8(j):
            return w8_ref[pl.ds(j * m_per, m_per), :]

        w_copy(0).start()
        w_copy(1).start()
        for d in range(N_DEV):
            with jax.named_scope(f"wconv#blk={d}"):
                w_copy(d).wait()
                w8_ref[pl.ds(jseq[d] * m_per, m_per), :] = w_vmem[
                    d % 2].astype(jnp.float8_e5m2)
                if d + 2 < N_DEV:
                    w_copy(d + 2).start()
            if d == 0:
                with jax.named_scope("local_dot"):
                    out_ref[...] = dot(recv_buf[my], w8(my))

        for d in range(1, N_DEV):
            src = jseq[d]
            for h in range(H):
                recv = pltpu.make_async_remote_copy(
                    src_ref=send_buf.at[0, pl.ds(0, m_h), :],
                    dst_ref=recv_buf.at[src, pl.ds(h * m_h, m_h), :],
                    send_sem=send_sems.at[0, 0],
                    recv_sem=recv_sems.at[src, h],
                    device_id=(my,),
                    device_id_type=pl.DeviceIdType.MESH,
                )
                with jax.named_scope(f"wait_recv#hop={d}_{h}"):
                    if _ABLATE != "nocomm":
                        recv.wait_recv()
                with jax.named_scope(f"dot#hop={d}_{h}"):
                    out_ref[pl.ds(h * m_h, m_h), :] += dot(
                        recv_buf[src, pl.ds(h * m_h, m_h), :], w8(src))

        with jax.named_scope("tail"):
            for rdma in rdmas:
                rdma.wait_send()

            out_ref[...] = out_ref[...] * (sx_ref[0] * sw_ref[0])

    return pl.pallas_call(
        body,
        out_shape=jax.ShapeDtypeStruct((m_per, n), jnp.float32),
        in_specs=[
            pl.BlockSpec(memory_space=pl.ANY),
            pl.BlockSpec(memory_space=pl.ANY),
            pl.BlockSpec(memory_space=pltpu.SMEM),
            pl.BlockSpec(memory_space=pltpu.SMEM),
        ],
        out_specs=pl.BlockSpec(memory_space=pltpu.VMEM),
        scratch_shapes=[
            pltpu.VMEM((4, m_h, k_shard), jnp.float32),
            pltpu.VMEM((N_DEV - 1, m_per, k_shard), jnp.float8_e4m3fn),
            pltpu.VMEM((N_DEV, m_per, k_shard), jnp.float8_e4m3fn),
            pltpu.VMEM((2, m_per, n), jnp.float32),
            pltpu.VMEM((k_total, n), jnp.float8_e5m2),
            pltpu.SemaphoreType.DMA((4,)),
            pltpu.SemaphoreType.DMA((N_DEV - 1, H)),
            pltpu.SemaphoreType.DMA((N_DEV, H)),
            pltpu.SemaphoreType.DMA((2,)),
        ],
        compiler_params=pltpu.CompilerParams(
            collective_id=0, vmem_limit_bytes=100 * 1024 * 1024),
    )(x, w_mat, scale_x, scale_w)
